# baseline (speedup 1.0000x reference)
import functools

import jax
import jax.numpy as jnp
from jax import lax
from jax.experimental import pallas as pl
from jax.experimental.pallas import tpu as pltpu
from jax.experimental.pallas import tpu_sc as plsc

LN_EPS = 1e-12

_NC = 2
_NS = 16
_L = 16
_NW = _NC * _NS


def _transform_body(x_ref, w_ref, b_ref, g_ref, beta_ref, o_ref):
    h = jnp.dot(x_ref[...], w_ref[...], preferred_element_type=jnp.float32)
    h = h + b_ref[...]
    h = 0.5 * h * (1.0 + lax.erf(h * (2.0 ** -0.5)))
    mu = jnp.mean(h, axis=-1, keepdims=True)
    var = jnp.mean((h - mu) ** 2, axis=-1, keepdims=True)
    h = (h - mu) * lax.rsqrt(var + LN_EPS) * g_ref[...] + beta_ref[...]
    o_ref[...] = h


def _transform(hidden_states, W_dense, b_dense, ln_gamma, ln_beta):
    B, H = hidden_states.shape
    D = W_dense.shape[1]
    BM = 512
    grid = (B // BM,)
    return pl.pallas_call(
        _transform_body,
        grid=grid,
        in_specs=[
            pl.BlockSpec((BM, H), lambda i: (i, 0)),
            pl.BlockSpec((H, D), lambda i: (0, 0)),
            pl.BlockSpec((1, D), lambda i: (0, 0)),
            pl.BlockSpec((1, D), lambda i: (0, 0)),
            pl.BlockSpec((1, D), lambda i: (0, 0)),
        ],
        out_specs=pl.BlockSpec((BM, D), lambda i: (i, 0)),
        out_shape=jax.ShapeDtypeStruct((B, D), jnp.float32),
    )(
        hidden_states,
        W_dense,
        b_dense.reshape(1, D),
        ln_gamma.reshape(1, D),
        ln_beta.reshape(1, D),
    )


def _pack_body(x_ref, o_ref):
    u = lax.bitcast_convert_type(x_ref[...], jnp.uint32)
    half = u.shape[1] // 2
    lo, hi = u[:, :half], u[:, half:]

    def rtne(b):
        return (b + jnp.uint32(0x7FFF) + ((b >> 16) & jnp.uint32(1))) >> 16

    packed = (rtne(hi) << 16) | rtne(lo)
    o_ref[...] = lax.bitcast_convert_type(packed, jnp.int32)


def _pack_table(table):
    V, D = table.shape
    BV = 1000
    grid = (V // BV,)
    return pl.pallas_call(
        _pack_body,
        grid=grid,
        in_specs=[pl.BlockSpec((BV, D), lambda i: (i, 0))],
        out_specs=pl.BlockSpec((BV, D // 2), lambda i: (i, 0)),
        out_shape=jax.ShapeDtypeStruct((V, D // 2), jnp.int32),
    )(table)


_GATHER_DNUMS = lax.GatherDimensionNumbers(
    offset_dims=(), collapsed_slice_dims=(0,), start_index_map=(0,))


def _shuffle(v, idx):
    return lax.gather(v, idx[:, None], _GATHER_DNUMS, (1,),
                      mode=lax.GatherScatterMode.PROMISE_IN_BOUNDS)


def _make_score_kernel(B, C, D, V):
    b_per_w = B // _NW
    n_chunks = D // _L
    n_groups = C // _L
    CH = 16
    NB = 4
    n_steps = b_per_w // NB
    steps_per_ch = CH // NB
    n_ch = b_per_w // CH
    mesh = plsc.VectorSubcoreMesh(core_axis_name="c", subcore_axis_name="s")

    @functools.partial(
        pl.kernel,
        mesh=mesh,
        compiler_params=pltpu.CompilerParams(needs_layout_passes=False),
        out_type=jax.ShapeDtypeStruct((B, C), jnp.float32),
        scratch_types=[
            pltpu.VMEM((2, CH, C), jnp.int32),
            pltpu.VMEM((2, CH, D), jnp.float32),
            pltpu.VMEM((NB, C, D // 2), jnp.int32),
            pltpu.VMEM((NB, C), jnp.float32),
            pltpu.VMEM((2, CH, C), jnp.float32),
            pltpu.SemaphoreType.DMA,
            pltpu.SemaphoreType.DMA,
            pltpu.SemaphoreType.DMA,
            pltpu.SemaphoreType.DMA,
            pltpu.SemaphoreType.DMA,
            pltpu.SemaphoreType.DMA,
        ],
    )
    def score_kernel(h_hbm, idx_hbm, table_hbm, bias_hbm, out_hbm,
                     idx_c, h_c, rows_v, bias_v, scores_v,
                     gsem0, gsem1, gsem2, gsem3, psem, ssem):
        wid = lax.axis_index("s") * _NC + lax.axis_index("c")
        base = wid * b_per_w
        lane = lax.iota(jnp.int32, _L)
        tree_rot = [(lane + (1 << s)) % _L for s in range(4)]
        tree_mask = [(lane & (1 << s)) == 0 for s in range(4)]
        gsems = (gsem0, gsem1, gsem2, gsem3)

        def issue_gather(off, p):
            idx_ref = idx_c.at[(off >> 4) & 1, off & (CH - 1)]
            pltpu.async_copy(table_hbm.at[idx_ref], rows_v.at[p], gsems[p])
            pltpu.async_copy(bias_hbm.at[idx_ref], bias_v.at[p], gsems[p])

        def wait_gather(p):
            idx_ref = idx_c.at[0, 0]
            pltpu.make_async_copy(
                table_hbm.at[idx_ref], rows_v.at[p], gsems[p]).wait()
            pltpu.make_async_copy(
                bias_hbm.at[idx_ref], bias_v.at[p], gsems[p]).wait()

        pltpu.sync_copy(idx_hbm.at[pl.ds(base, CH)], idx_c.at[0])
        pltpu.sync_copy(h_hbm.at[pl.ds(base, CH)], h_c.at[0])
        for p in range(NB):
            issue_gather(p, p)

        def step_body(t, carry):
            s = t >> 2

            @pl.when((t & (steps_per_ch - 1)) == 0)
            def _prefetch_next_chunk():
                s_next = jnp.minimum(s + 1, n_ch - 1)
                q = (s + 1) & 1
                pltpu.async_copy(
                    idx_hbm.at[pl.ds(base + s_next * CH, CH)],
                    idx_c.at[q], psem)
                pltpu.async_copy(
                    h_hbm.at[pl.ds(base + s_next * CH, CH)],
                    h_c.at[q], psem)

            @pl.when(((t & (steps_per_ch - 1)) == 0) & (s >= 2))
            def _wait_prev_score_store():
                pltpu.make_async_copy(
                    scores_v.at[s & 1],
                    out_hbm.at[pl.ds(base + (s - 2) * CH, CH)], ssem).wait()

            @pl.when((t & (steps_per_ch - 1)) == steps_per_ch - 1)
            def _wait_next_chunk():
                pltpu.make_async_copy(
                    idx_hbm.at[pl.ds(base, CH)], idx_c.at[0], psem).wait()
                pltpu.make_async_copy(
                    h_hbm.at[pl.ds(base, CH)], h_c.at[0], psem).wait()

            sbuf = s & 1
            for p in range(NB):
                off = NB * t + p
                r = off & (CH - 1)
                wait_gather(p)
                hs = [h_c[sbuf, r, pl.ds(k * _L, _L)]
                      for k in range(n_chunks)]

                def cgroup(g, _p=p, _r=r, _hs=hs, _sb=s & 1):
                    c0 = g * _L
                    accs = []
                    nk = n_chunks // 2
                    for j in range(_L):
                        w = rows_v[_p, c0 + j, pl.ds(0, _L)]
                        acc_lo = plsc.bitcast(
                            lax.shift_left(w, 16), jnp.float32) * _hs[0]
                        acc_hi = plsc.bitcast(w, jnp.float32) * _hs[nk]
                        for k in range(1, nk):
                            w = rows_v[_p, c0 + j, pl.ds(k * _L, _L)]
                            acc_lo = acc_lo + plsc.bitcast(
                                lax.shift_left(w, 16), jnp.float32) * _hs[k]
                            acc_hi = acc_hi + plsc.bitcast(
                                w, jnp.float32) * _hs[nk + k]
                        accs.append(acc_lo + acc_hi)
                    for s in range(4):
                        nxt = []
                        for gg in range(len(accs) // 2):
                            a, b = accs[2 * gg], accs[2 * gg + 1]
                            sel_e = jnp.where(tree_mask[s], a, b)
                            sel_o = jnp.where(tree_mask[s], b, a)
                            nxt.append(sel_e + _shuffle(sel_o, tree_rot[s]))
                        accs = nxt
                    scores_v[_sb, _r, pl.ds(c0, _L)] = (
                        accs[0] + bias_v[_p, pl.ds(c0, _L)])

                plsc.parallel_loop(0, n_groups)(cgroup)
                issue_gather(jnp.minimum(off + NB, b_per_w - 1), p)

            @pl.when((t & (steps_per_ch - 1)) == steps_per_ch - 1)
            def _store_scores():
                pltpu.async_copy(scores_v.at[s & 1],
                                 out_hbm.at[pl.ds(base + s * CH, CH)], ssem)

            return carry

        lax.fori_loop(0, n_steps, step_body, 0)
        for p in range(NB):
            wait_gather(p)
        for s_tail in (n_ch - 2, n_ch - 1):
            pltpu.make_async_copy(
                scores_v.at[s_tail & 1],
                out_hbm.at[pl.ds(base + s_tail * CH, CH)], ssem).wait()

    return score_kernel


def kernel(hidden_states, cand_emb_index, W_dense, b_dense, ln_gamma,
           ln_beta, decoder_table, entity_bias):
    B, H = hidden_states.shape
    V, D = decoder_table.shape
    C = cand_emb_index.shape[1]
    idx = cand_emb_index.astype(jnp.int32)
    h = _transform(hidden_states, W_dense, b_dense, ln_gamma, ln_beta)
    table_packed = _pack_table(decoder_table)
    score = _make_score_kernel(B, C, D, V)
    return score(h, idx, table_packed, entity_bias)


# speedup vs baseline: 2.3140x; 2.3140x over previous
import functools

import jax
import jax.numpy as jnp
from jax import lax
from jax.experimental import pallas as pl
from jax.experimental.pallas import tpu as pltpu
from jax.experimental.pallas import tpu_sc as plsc

LN_EPS = 1e-12

_NC = 2
_NS = 16
_L = 16
_NW = _NC * _NS


def _transform_body(x_ref, w_ref, b_ref, g_ref, beta_ref, o_ref):
    h = jnp.dot(x_ref[...], w_ref[...], preferred_element_type=jnp.float32)
    h = h + b_ref[...]
    h = 0.5 * h * (1.0 + lax.erf(h * (2.0 ** -0.5)))
    mu = jnp.mean(h, axis=-1, keepdims=True)
    var = jnp.mean((h - mu) ** 2, axis=-1, keepdims=True)
    h = (h - mu) * lax.rsqrt(var + LN_EPS) * g_ref[...] + beta_ref[...]
    o_ref[...] = h


def _transform(hidden_states, W_dense, b_dense, ln_gamma, ln_beta):
    B, H = hidden_states.shape
    D = W_dense.shape[1]
    BM = 512
    grid = (B // BM,)
    return pl.pallas_call(
        _transform_body,
        grid=grid,
        in_specs=[
            pl.BlockSpec((BM, H), lambda i: (i, 0)),
            pl.BlockSpec((H, D), lambda i: (0, 0)),
            pl.BlockSpec((1, D), lambda i: (0, 0)),
            pl.BlockSpec((1, D), lambda i: (0, 0)),
            pl.BlockSpec((1, D), lambda i: (0, 0)),
        ],
        out_specs=pl.BlockSpec((BM, D), lambda i: (i, 0)),
        out_shape=jax.ShapeDtypeStruct((B, D), jnp.float32),
    )(
        hidden_states,
        W_dense,
        b_dense.reshape(1, D),
        ln_gamma.reshape(1, D),
        ln_beta.reshape(1, D),
    )


def _pack_body(x_ref, o_ref):
    u = lax.bitcast_convert_type(x_ref[...], jnp.uint32)
    half = u.shape[1] // 2
    lo, hi = u[:, :half], u[:, half:]

    def rtne(b):
        return (b + jnp.uint32(0x7FFF) + ((b >> 16) & jnp.uint32(1))) >> 16

    packed = (rtne(hi) << 16) | rtne(lo)
    o_ref[...] = lax.bitcast_convert_type(packed, jnp.int32)


def _pack_table(table):
    V, D = table.shape
    BV = 1000
    grid = (V // BV,)
    return pl.pallas_call(
        _pack_body,
        grid=grid,
        in_specs=[pl.BlockSpec((BV, D), lambda i: (i, 0))],
        out_specs=pl.BlockSpec((BV, D // 2), lambda i: (i, 0)),
        out_shape=jax.ShapeDtypeStruct((V, D // 2), jnp.int32),
    )(table)


_GATHER_DNUMS = lax.GatherDimensionNumbers(
    offset_dims=(), collapsed_slice_dims=(0,), start_index_map=(0,))


def _shuffle(v, idx):
    return lax.gather(v, idx[:, None], _GATHER_DNUMS, (1,),
                      mode=lax.GatherScatterMode.PROMISE_IN_BOUNDS)


def _make_score_kernel(B, C, D, V):
    b_per_w = B // _NW
    n_chunks = D // _L
    n_groups = C // _L
    CH = 16
    NB = 4
    n_steps = b_per_w // NB
    steps_per_ch = CH // NB
    n_ch = b_per_w // CH
    mesh = plsc.VectorSubcoreMesh(core_axis_name="c", subcore_axis_name="s")

    @functools.partial(
        pl.kernel,
        mesh=mesh,
        compiler_params=pltpu.CompilerParams(needs_layout_passes=False),
        out_type=jax.ShapeDtypeStruct((B, C), jnp.float32),
        scratch_types=[
            pltpu.VMEM((2, CH, C), jnp.int32),
            pltpu.VMEM((2, CH, D), jnp.float32),
            pltpu.VMEM((NB, C, D // 2), jnp.int32),
            pltpu.VMEM((NB, C), jnp.float32),
            pltpu.VMEM((2, CH, C), jnp.float32),
            pltpu.SemaphoreType.DMA,
            pltpu.SemaphoreType.DMA,
            pltpu.SemaphoreType.DMA,
            pltpu.SemaphoreType.DMA,
            pltpu.SemaphoreType.DMA,
            pltpu.SemaphoreType.DMA,
        ],
    )
    def score_kernel(h_hbm, idx_hbm, table_hbm, bias_hbm, out_hbm,
                     idx_c, h_c, rows_v, bias_v, scores_v,
                     gsem0, gsem1, gsem2, gsem3, psem, ssem):
        wid = lax.axis_index("s") * _NC + lax.axis_index("c")
        base = wid * b_per_w
        lane = lax.iota(jnp.int32, _L)
        tree_rot = [(lane + (1 << s)) % _L for s in range(4)]
        tree_mask = [(lane & (1 << s)) == 0 for s in range(4)]
        gsems = (gsem0, gsem1, gsem2, gsem3)

        def issue_gather(off, p):
            idx_ref = idx_c.at[(off >> 4) & 1, off & (CH - 1)]
            pltpu.async_copy(table_hbm.at[idx_ref], rows_v.at[p], gsems[p])
            pltpu.async_copy(bias_hbm.at[idx_ref], bias_v.at[p], gsems[p])

        def wait_gather(p):
            idx_ref = idx_c.at[0, 0]
            pltpu.make_async_copy(
                table_hbm.at[idx_ref], rows_v.at[p], gsems[p]).wait()
            pltpu.make_async_copy(
                bias_hbm.at[idx_ref], bias_v.at[p], gsems[p]).wait()

        pltpu.sync_copy(idx_hbm.at[pl.ds(base, CH)], idx_c.at[0])
        pltpu.sync_copy(h_hbm.at[pl.ds(base, CH)], h_c.at[0])
        for p in range(NB):
            issue_gather(p, p)

        def step_body(t, carry):
            s = t >> 2

            @pl.when((t & (steps_per_ch - 1)) == 0)
            def _prefetch_next_chunk():
                s_next = jnp.minimum(s + 1, n_ch - 1)
                q = (s + 1) & 1
                pltpu.async_copy(
                    idx_hbm.at[pl.ds(base + s_next * CH, CH)],
                    idx_c.at[q], psem)
                pltpu.async_copy(
                    h_hbm.at[pl.ds(base + s_next * CH, CH)],
                    h_c.at[q], psem)

            @pl.when(((t & (steps_per_ch - 1)) == 0) & (s >= 2))
            def _wait_prev_score_store():
                pltpu.make_async_copy(
                    scores_v.at[s & 1],
                    out_hbm.at[pl.ds(base + (s - 2) * CH, CH)], ssem).wait()

            @pl.when((t & (steps_per_ch - 1)) == steps_per_ch - 1)
            def _wait_next_chunk():
                pltpu.make_async_copy(
                    idx_hbm.at[pl.ds(base, CH)], idx_c.at[0], psem).wait()
                pltpu.make_async_copy(
                    h_hbm.at[pl.ds(base, CH)], h_c.at[0], psem).wait()

            sbuf = s & 1
            for p in range(NB):
                off = NB * t + p
                r = off & (CH - 1)
                wait_gather(p)
                hs = [h_c[sbuf, r, pl.ds(k * _L, _L)]
                      for k in range(n_chunks)]

                def cgroup(g, carry2, _p=p, _r=r, _hs=hs, _sb=s & 1):
                    c0 = g * _L
                    accs = []
                    nk = n_chunks // 2
                    for j in range(_L):
                        w = rows_v[_p, c0 + j, pl.ds(0, _L)]
                        acc_lo = plsc.bitcast(
                            lax.shift_left(w, 16), jnp.float32) * _hs[0]
                        acc_hi = plsc.bitcast(w, jnp.float32) * _hs[nk]
                        for k in range(1, nk):
                            w = rows_v[_p, c0 + j, pl.ds(k * _L, _L)]
                            acc_lo = acc_lo + plsc.bitcast(
                                lax.shift_left(w, 16), jnp.float32) * _hs[k]
                            acc_hi = acc_hi + plsc.bitcast(
                                w, jnp.float32) * _hs[nk + k]
                        accs.append(acc_lo + acc_hi)
                    for s in range(4):
                        nxt = []
                        for gg in range(len(accs) // 2):
                            a, b = accs[2 * gg], accs[2 * gg + 1]
                            sel_e = jnp.where(tree_mask[s], a, b)
                            sel_o = jnp.where(tree_mask[s], b, a)
                            nxt.append(sel_e + _shuffle(sel_o, tree_rot[s]))
                        accs = nxt
                    scores_v[_sb, _r, pl.ds(c0, _L)] = (
                        accs[0] + bias_v[_p, pl.ds(c0, _L)])
                    return carry2

                lax.fori_loop(0, n_groups, cgroup, 0)
                issue_gather(jnp.minimum(off + NB, b_per_w - 1), p)

            @pl.when((t & (steps_per_ch - 1)) == steps_per_ch - 1)
            def _store_scores():
                pltpu.async_copy(scores_v.at[s & 1],
                                 out_hbm.at[pl.ds(base + s * CH, CH)], ssem)

            return carry

        lax.fori_loop(0, n_steps, step_body, 0)
        for p in range(NB):
            wait_gather(p)
        for s_tail in (n_ch - 2, n_ch - 1):
            pltpu.make_async_copy(
                scores_v.at[s_tail & 1],
                out_hbm.at[pl.ds(base + s_tail * CH, CH)], ssem).wait()

    return score_kernel


def kernel(hidden_states, cand_emb_index, W_dense, b_dense, ln_gamma,
           ln_beta, decoder_table, entity_bias):
    B, H = hidden_states.shape
    V, D = decoder_table.shape
    C = cand_emb_index.shape[1]
    idx = cand_emb_index.astype(jnp.int32)
    h = _transform(hidden_states, W_dense, b_dense, ln_gamma, ln_beta)
    table_packed = _pack_table(decoder_table)
    score = _make_score_kernel(B, C, D, V)
    return score(h, idx, table_packed, entity_bias)
